# baseline (speedup 1.0000x reference)
import functools

import jax
import jax.numpy as jnp
from jax.experimental import pallas as pl
from jax.experimental.pallas import tpu as pltpu

_B, _T, _D = 4, 4096, 2048
_NE = 16
_THRESH = 0.15
_NEG = -1e30


def _router_body(x_ref, w_ref, im_ref, rw_ref, idx_ref, aux_ref, cnt_ref,
                 *, nblocks, tb):
    i = pl.program_id(0)

    xb = x_ref[...]
    scores = jax.lax.dot_general(
        xb.astype(jnp.bfloat16), w_ref[...].astype(jnp.bfloat16),
        (((1,), (0,)), ((), ())),
        preferred_element_type=jnp.float32)
    es = scores[:, :_NE]
    g0 = jax.nn.sigmoid(scores[:, _NE:_NE + 1])
    g1 = jax.nn.sigmoid(scores[:, _NE + 1:_NE + 2])

    lane = jax.lax.broadcasted_iota(jnp.int32, (tb, _NE), 1)
    mask_a = lane < 8
    mask_b = jnp.logical_and(lane >= 8, lane < 12)
    mask_c = lane >= 12

    sm_a = jnp.where(mask_a, es, _NEG)
    m_a = jnp.max(sm_a, axis=-1, keepdims=True)
    sm_b = jnp.where(mask_b, es, _NEG)
    m_b = jnp.max(sm_b, axis=-1, keepdims=True)
    sm_c = jnp.where(mask_c, es, _NEG)
    m_c = jnp.max(sm_c, axis=-1, keepdims=True)

    mbar = jnp.where(mask_a, m_a, jnp.where(mask_b, m_b, m_c))
    e_all = jnp.exp(es - mbar)
    zero = jnp.zeros((), jnp.float32)
    z_a = jnp.sum(jnp.where(mask_a, e_all, zero), axis=-1, keepdims=True)
    z_b = jnp.sum(jnp.where(mask_b, e_all, zero), axis=-1, keepdims=True)
    z_c = jnp.sum(jnp.where(mask_c, e_all, zero), axis=-1, keepdims=True)

    one = jnp.ones((), jnp.float32)
    oh_a = jnp.where(sm_a == m_a, one, zero)
    oh_b = jnp.where(sm_b == m_b, one, zero)
    oh_c1 = jnp.where(sm_c == m_c, one, zero)
    sm_c2 = jnp.where(sm_c == m_c, _NEG, sm_c)
    m_c2 = jnp.max(sm_c2, axis=-1, keepdims=True)
    oh_c2 = jnp.where(sm_c2 == m_c2, one, zero)

    oh64 = jnp.concatenate([oh_a, oh_b, oh_c1, oh_c2], axis=-1)
    idx4 = jax.lax.dot_general(
        oh64.astype(jnp.bfloat16), im_ref[...].astype(jnp.bfloat16),
        (((1,), (0,)), ((), ())),
        preferred_element_type=jnp.float32)

    w0 = 1.0 / z_a
    w1 = (1.0 / z_b) * g0 * (g0 > _THRESH).astype(jnp.float32)
    gate_c = g1 * (g1 > _THRESH).astype(jnp.float32)
    w2 = (1.0 / z_c) * gate_c
    w3 = (jnp.exp(m_c2 - m_c) / z_c) * gate_c

    zeros2 = jnp.zeros((tb, 2), jnp.float32)
    rw = jnp.concatenate([w0, w1, w2, w3, zeros2], axis=-1)
    rw = rw / (jnp.sum(rw, axis=-1, keepdims=True) + 1e-8)
    rw_ref[...] = rw
    izeros2 = jnp.zeros((tb, 2), jnp.int32)
    idx_ref[...] = jnp.concatenate(
        [idx4[:, 0:4].astype(jnp.int32), izeros2], axis=-1)

    @pl.when(i == 0)
    def _():
        cnt_ref[...] = jnp.zeros_like(cnt_ref)

    slotsum = oh_a + oh_b + oh_c1 + oh_c2
    ones_row = jnp.ones((8, tb), jnp.bfloat16)
    bc = jax.lax.dot_general(
        ones_row, slotsum.astype(jnp.bfloat16),
        (((1,), (0,)), ((), ())),
        preferred_element_type=jnp.float32)
    cnt_ref[0:1, 0:_NE] += bc[0:1, :]

    @pl.when(i == nblocks - 1)
    def _():
        lane1 = jax.lax.broadcasted_iota(jnp.int32, (1, _NE), 1)
        pad = jnp.where(lane1 == 0, jnp.float32(2 * _B * _T), 0.0)
        counts = cnt_ref[0:1, 0:_NE] + pad
        total = jnp.sum(counts)
        log_u = jnp.log(jnp.float32(1.0 / _NE))
        aux = (0.01 / _NE) * jnp.sum(log_u - jnp.log(counts / total),
                                     axis=-1, keepdims=True)
        aux_ref[...] = aux


def _index_matrix():
    m = [[0.0] * 8 for _ in range(64)]
    for blk in range(4):
        for e in range(_NE):
            m[blk * _NE + e][blk] = float(e)
    return m


@functools.partial(jax.jit, static_argnames=("tb",))
def _run(x, W_expert, W_group, tb=1024):
    n = _B * _T
    nblocks = n // tb
    xf = x.reshape(n, _D)
    w = jnp.concatenate([W_expert, W_group], axis=0).T
    im = jnp.asarray(_index_matrix(), jnp.float32)

    rw, idx, aux = pl.pallas_call(
        functools.partial(_router_body, nblocks=nblocks, tb=tb),
        grid=(nblocks,),
        in_specs=[
            pl.BlockSpec((tb, _D), lambda i: (i, 0)),
            pl.BlockSpec((_D, _NE + 2), lambda i: (0, 0)),
            pl.BlockSpec((64, 8), lambda i: (0, 0)),
        ],
        out_specs=[
            pl.BlockSpec((tb, 6), lambda i: (i, 0)),
            pl.BlockSpec((tb, 6), lambda i: (i, 0)),
            pl.BlockSpec((1, 1), lambda i: (0, 0)),
        ],
        out_shape=[
            jax.ShapeDtypeStruct((n, 6), jnp.float32),
            jax.ShapeDtypeStruct((n, 6), jnp.int32),
            jax.ShapeDtypeStruct((1, 1), jnp.float32),
        ],
        scratch_shapes=[pltpu.VMEM((8, 128), jnp.float32)],
        compiler_params=pltpu.CompilerParams(
            dimension_semantics=("arbitrary",)),
    )(xf, w, im)

    return (rw.reshape(_B, _T, 6), idx.reshape(_B, _T, 6), aux[0, 0])


def kernel(x, W_expert, W_group):
    return _run(x, W_expert, W_group)


# speedup vs baseline: 1.0059x; 1.0059x over previous
import functools

import jax
import jax.numpy as jnp
from jax.experimental import pallas as pl
from jax.experimental.pallas import tpu as pltpu

_B, _T, _D = 4, 4096, 2048
_NE = 16
_THRESH = 0.15
_NEG = -1e30


def _router_body(x_ref, w_ref, im_ref, rw_ref, idx_ref, aux_ref, cnt_ref,
                 *, nblocks, tb):
    i = pl.program_id(0)

    xb = x_ref[...]
    scores = jax.lax.dot_general(
        xb, w_ref[...], (((1,), (0,)), ((), ())),
        preferred_element_type=jnp.float32)
    es = scores[:, :_NE]
    g0 = jax.nn.sigmoid(scores[:, _NE:_NE + 1])
    g1 = jax.nn.sigmoid(scores[:, _NE + 1:_NE + 2])

    lane = jax.lax.broadcasted_iota(jnp.int32, (tb, _NE), 1)
    mask_a = lane < 8
    mask_b = jnp.logical_and(lane >= 8, lane < 12)
    mask_c = lane >= 12

    sm_a = jnp.where(mask_a, es, _NEG)
    m_a = jnp.max(sm_a, axis=-1, keepdims=True)
    sm_b = jnp.where(mask_b, es, _NEG)
    m_b = jnp.max(sm_b, axis=-1, keepdims=True)
    sm_c = jnp.where(mask_c, es, _NEG)
    m_c = jnp.max(sm_c, axis=-1, keepdims=True)

    mbar = jnp.where(mask_a, m_a, jnp.where(mask_b, m_b, m_c))
    e_all = jnp.exp(es - mbar)
    zero = jnp.zeros((), jnp.float32)
    z_a = jnp.sum(jnp.where(mask_a, e_all, zero), axis=-1, keepdims=True)
    z_b = jnp.sum(jnp.where(mask_b, e_all, zero), axis=-1, keepdims=True)
    z_c = jnp.sum(jnp.where(mask_c, e_all, zero), axis=-1, keepdims=True)

    one = jnp.ones((), jnp.float32)
    oh_a = jnp.where(sm_a == m_a, one, zero)
    oh_b = jnp.where(sm_b == m_b, one, zero)
    oh_c1 = jnp.where(sm_c == m_c, one, zero)
    sm_c2 = jnp.where(sm_c == m_c, _NEG, sm_c)
    m_c2 = jnp.max(sm_c2, axis=-1, keepdims=True)
    oh_c2 = jnp.where(sm_c2 == m_c2, one, zero)

    oh64 = jnp.concatenate([oh_a, oh_b, oh_c1, oh_c2], axis=-1)
    idx4 = jax.lax.dot_general(
        oh64, im_ref[...], (((1,), (0,)), ((), ())),
        preferred_element_type=jnp.float32)

    w0 = 1.0 / z_a
    w1 = (1.0 / z_b) * g0 * (g0 > _THRESH).astype(jnp.float32)
    gate_c = g1 * (g1 > _THRESH).astype(jnp.float32)
    w2 = (1.0 / z_c) * gate_c
    w3 = (jnp.exp(m_c2 - m_c) / z_c) * gate_c

    zeros2 = jnp.zeros((tb, 2), jnp.float32)
    rw = jnp.concatenate([w0, w1, w2, w3, zeros2], axis=-1)
    rw = rw / (jnp.sum(rw, axis=-1, keepdims=True) + 1e-8)
    rw_ref[...] = rw
    izeros2 = jnp.zeros((tb, 2), jnp.int32)
    idx_ref[...] = jnp.concatenate(
        [idx4[:, 0:4].astype(jnp.int32), izeros2], axis=-1)

    @pl.when(i == 0)
    def _():
        cnt_ref[...] = jnp.zeros_like(cnt_ref)

    slotsum = oh_a + oh_b + oh_c1 + oh_c2
    ones_row = jnp.ones((8, tb), jnp.float32)
    bc = jax.lax.dot_general(
        ones_row, slotsum, (((1,), (0,)), ((), ())),
        preferred_element_type=jnp.float32)
    cnt_ref[0:1, 0:_NE] += bc[0:1, :]

    @pl.when(i == nblocks - 1)
    def _():
        lane1 = jax.lax.broadcasted_iota(jnp.int32, (1, _NE), 1)
        pad = jnp.where(lane1 == 0, jnp.float32(2 * _B * _T), 0.0)
        counts = cnt_ref[0:1, 0:_NE] + pad
        total = jnp.sum(counts)
        log_u = jnp.log(jnp.float32(1.0 / _NE))
        aux = (0.01 / _NE) * jnp.sum(log_u - jnp.log(counts / total),
                                     axis=-1, keepdims=True)
        aux_ref[...] = aux


def _index_matrix():
    m = [[0.0] * 8 for _ in range(64)]
    for blk in range(4):
        for e in range(_NE):
            m[blk * _NE + e][blk] = float(e)
    return m


@functools.partial(jax.jit, static_argnames=("tb",))
def _run(x, W_expert, W_group, tb=1024):
    n = _B * _T
    nblocks = n // tb
    xf = x.reshape(n, _D)
    w = jnp.concatenate([W_expert, W_group], axis=0).T
    im = jnp.asarray(_index_matrix(), jnp.float32)

    rw, idx, aux = pl.pallas_call(
        functools.partial(_router_body, nblocks=nblocks, tb=tb),
        grid=(nblocks,),
        in_specs=[
            pl.BlockSpec((tb, _D), lambda i: (i, 0)),
            pl.BlockSpec((_D, _NE + 2), lambda i: (0, 0)),
            pl.BlockSpec((64, 8), lambda i: (0, 0)),
        ],
        out_specs=[
            pl.BlockSpec((tb, 6), lambda i: (i, 0)),
            pl.BlockSpec((tb, 6), lambda i: (i, 0)),
            pl.BlockSpec((1, 1), lambda i: (0, 0)),
        ],
        out_shape=[
            jax.ShapeDtypeStruct((n, 6), jnp.float32),
            jax.ShapeDtypeStruct((n, 6), jnp.int32),
            jax.ShapeDtypeStruct((1, 1), jnp.float32),
        ],
        scratch_shapes=[pltpu.VMEM((8, 128), jnp.float32)],
        compiler_params=pltpu.CompilerParams(
            dimension_semantics=("arbitrary",)),
    )(xf, w, im)

    return (rw.reshape(_B, _T, 6), idx.reshape(_B, _T, 6), aux[0, 0])


def kernel(x, W_expert, W_group):
    return _run(x, W_expert, W_group)


# speedup vs baseline: 1.1477x; 1.1410x over previous
import functools

import jax
import jax.numpy as jnp
from jax.experimental import pallas as pl
from jax.experimental.pallas import tpu as pltpu

_B, _T, _D = 4, 4096, 2048
_NE = 16
_THRESH = 0.15
_NEG = -1e30


def _router_body(x_ref, w_ref, rw_ref, idx_ref, aux_ref, cnt_ref, *, nblocks, tb):
    i = pl.program_id(0)

    xb = x_ref[...]
    scores = jax.lax.dot_general(
        xb, w_ref[...], (((1,), (0,)), ((), ())),
        preferred_element_type=jnp.float32)
    es = scores[:, :_NE]
    g0 = jax.nn.sigmoid(scores[:, _NE:_NE + 1])
    g1 = jax.nn.sigmoid(scores[:, _NE + 1:_NE + 2])

    lane = jax.lax.broadcasted_iota(jnp.int32, (tb, _NE), 1)
    mask_a = lane < 8
    mask_b = jnp.logical_and(lane >= 8, lane < 12)
    mask_c = lane >= 12

    def top1(mask, s):
        sm = jnp.where(mask, s, _NEG)
        m = jnp.max(sm, axis=-1, keepdims=True)
        idx = jnp.min(jnp.where(sm == m, lane, _NE), axis=-1, keepdims=True)
        z = jnp.sum(jnp.where(mask, jnp.exp(s - m), 0.0), axis=-1, keepdims=True)
        return m, idx, z

    m_a, idx_a, z_a = top1(mask_a, es)
    p_a = 1.0 / z_a

    m_b, idx_b, z_b = top1(mask_b, es)
    w_b = (1.0 / z_b) * g0 * (g0 > _THRESH).astype(jnp.float32)

    m_c, idx_c1, z_c = top1(mask_c, es)
    p_c1 = 1.0 / z_c
    mask_c2 = jnp.logical_and(mask_c, lane != idx_c1)
    sm2 = jnp.where(mask_c2, es, _NEG)
    m_c2 = jnp.max(sm2, axis=-1, keepdims=True)
    idx_c2 = jnp.min(jnp.where(sm2 == m_c2, lane, _NE), axis=-1, keepdims=True)
    p_c2 = jnp.exp(m_c2 - m_c) / z_c
    gate_c = g1 * (g1 > _THRESH).astype(jnp.float32)
    w_c1 = p_c1 * gate_c
    w_c2 = p_c2 * gate_c

    zeros = jnp.zeros((tb, 2), jnp.float32)
    rw = jnp.concatenate([p_a, w_b, w_c1, w_c2, zeros], axis=-1)
    rw = rw / (jnp.sum(rw, axis=-1, keepdims=True) + 1e-8)
    rw_ref[...] = rw
    izeros = jnp.zeros((tb, 2), jnp.int32)
    idx_ref[...] = jnp.concatenate([idx_a, idx_b, idx_c1, idx_c2, izeros],
                                   axis=-1)

    bc = jnp.zeros((1, _NE), jnp.float32)
    for idx in (idx_a, idx_b, idx_c1, idx_c2):
        oh = (jnp.broadcast_to(idx, (tb, _NE)) == lane).astype(jnp.float32)
        bc = bc + jnp.sum(oh, axis=0, keepdims=True)

    @pl.when(i == 0)
    def _():
        cnt_ref[...] = jnp.zeros_like(cnt_ref)

    cnt_ref[0:1, 0:_NE] += bc

    @pl.when(i == nblocks - 1)
    def _():
        lane1 = jax.lax.broadcasted_iota(jnp.int32, (1, _NE), 1)
        pad = jnp.where(lane1 == 0, jnp.float32(2 * _B * _T), 0.0)
        counts = cnt_ref[0:1, 0:_NE] + pad
        total = jnp.sum(counts)
        log_u = jnp.log(jnp.float32(1.0 / _NE))
        aux = (0.01 / _NE) * jnp.sum(log_u - jnp.log(counts / total),
                                     axis=-1, keepdims=True)
        aux_ref[...] = aux


@functools.partial(jax.jit, static_argnames=("tb",))
def _run(x, W_expert, W_group, tb=1024):
    n = _B * _T
    nblocks = n // tb
    xf = x.reshape(n, _D)
    w = jnp.concatenate([W_expert, W_group], axis=0).T

    rw, idx, aux = pl.pallas_call(
        functools.partial(_router_body, nblocks=nblocks, tb=tb),
        grid=(nblocks,),
        in_specs=[
            pl.BlockSpec((tb, _D), lambda i: (i, 0)),
            pl.BlockSpec((_D, _NE + 2), lambda i: (0, 0)),
        ],
        out_specs=[
            pl.BlockSpec((tb, 6), lambda i: (i, 0)),
            pl.BlockSpec((tb, 6), lambda i: (i, 0)),
            pl.BlockSpec((1, 1), lambda i: (0, 0)),
        ],
        out_shape=[
            jax.ShapeDtypeStruct((n, 6), jnp.float32),
            jax.ShapeDtypeStruct((n, 6), jnp.int32),
            jax.ShapeDtypeStruct((1, 1), jnp.float32),
        ],
        scratch_shapes=[pltpu.VMEM((8, 128), jnp.float32)],
        compiler_params=pltpu.CompilerParams(
            dimension_semantics=("arbitrary",)),
    )(xf, w)

    return (rw.reshape(_B, _T, 6), idx.reshape(_B, _T, 6), aux[0, 0])


def kernel(x, W_expert, W_group):
    return _run(x, W_expert, W_group)


# speedup vs baseline: 1.2652x; 1.1023x over previous
import functools

import jax
import jax.numpy as jnp
from jax.experimental import pallas as pl
from jax.experimental.pallas import tpu as pltpu

_B, _T, _D = 4, 4096, 2048
_NE = 16
_THRESH = 0.15
_NEG = -1e30


def _router_body(x_ref, w_ref, rw_ref, idx_ref, aux_ref, cnt_ref, *, nblocks, tb):
    i = pl.program_id(0)

    xb = x_ref[...]
    scores = jax.lax.dot_general(
        xb, w_ref[...], (((1,), (0,)), ((), ())),
        preferred_element_type=jnp.float32)
    es = scores[:, :_NE]
    g0 = jax.nn.sigmoid(scores[:, _NE:_NE + 1])
    g1 = jax.nn.sigmoid(scores[:, _NE + 1:_NE + 2])

    lane = jax.lax.broadcasted_iota(jnp.int32, (tb, _NE), 1)
    lane_f = lane.astype(jnp.float32)
    mask_a = lane < 8
    mask_b = jnp.logical_and(lane >= 8, lane < 12)
    mask_c = lane >= 12

    def top1(mask, s):
        sm = jnp.where(mask, s, _NEG)
        m = jnp.max(sm, axis=-1, keepdims=True)
        idx_f = jnp.min(jnp.where(sm == m, lane_f, 16.0), axis=-1, keepdims=True)
        z = jnp.sum(jnp.where(mask, jnp.exp(s - m), 0.0), axis=-1, keepdims=True)
        return m, idx_f.astype(jnp.int32), z

    m_a, idx_a, z_a = top1(mask_a, es)
    p_a = 1.0 / z_a

    m_b, idx_b, z_b = top1(mask_b, es)
    w_b = (1.0 / z_b) * g0 * (g0 > _THRESH).astype(jnp.float32)

    m_c, idx_c1, z_c = top1(mask_c, es)
    p_c1 = 1.0 / z_c
    mask_c2 = jnp.logical_and(mask_c, lane != idx_c1)
    sm2 = jnp.where(mask_c2, es, _NEG)
    m_c2 = jnp.max(sm2, axis=-1, keepdims=True)
    idx_c2 = jnp.min(jnp.where(sm2 == m_c2, lane_f, 16.0), axis=-1,
                     keepdims=True).astype(jnp.int32)
    p_c2 = jnp.exp(m_c2 - m_c) / z_c
    gate_c = g1 * (g1 > _THRESH).astype(jnp.float32)
    w_c1 = p_c1 * gate_c
    w_c2 = p_c2 * gate_c

    zeros = jnp.zeros((tb, 2), jnp.float32)
    rw = jnp.concatenate([p_a, w_b, w_c1, w_c2, zeros], axis=-1)
    rw = rw / (jnp.sum(rw, axis=-1, keepdims=True) + 1e-8)
    rw_ref[...] = rw
    izeros = jnp.zeros((tb, 2), jnp.int32)
    idx_ref[...] = jnp.concatenate([idx_a, idx_b, idx_c1, idx_c2, izeros],
                                   axis=-1)

    bc = jnp.zeros((1, _NE), jnp.float32)
    for idx in (idx_a, idx_b, idx_c1, idx_c2):
        oh = (jnp.broadcast_to(idx, (tb, _NE)) == lane).astype(jnp.float32)
        bc = bc + jnp.sum(oh, axis=0, keepdims=True)

    @pl.when(i == 0)
    def _():
        cnt_ref[...] = jnp.zeros_like(cnt_ref)

    cnt_ref[0:1, 0:_NE] += bc

    @pl.when(i == nblocks - 1)
    def _():
        lane1 = jax.lax.broadcasted_iota(jnp.int32, (1, _NE), 1)
        pad = jnp.where(lane1 == 0, jnp.float32(2 * _B * _T), 0.0)
        counts = cnt_ref[0:1, 0:_NE] + pad
        total = jnp.sum(counts)
        log_u = jnp.log(jnp.float32(1.0 / _NE))
        aux = (0.01 / _NE) * jnp.sum(log_u - jnp.log(counts / total),
                                     axis=-1, keepdims=True)
        aux_ref[...] = aux


@functools.partial(jax.jit, static_argnames=("tb",))
def _run(x, W_expert, W_group, tb=1024):
    n = _B * _T
    nblocks = n // tb
    xf = x.reshape(n, _D)
    w = jnp.concatenate([W_expert, W_group], axis=0).T

    rw, idx, aux = pl.pallas_call(
        functools.partial(_router_body, nblocks=nblocks, tb=tb),
        grid=(nblocks,),
        in_specs=[
            pl.BlockSpec((tb, _D), lambda i: (i, 0)),
            pl.BlockSpec((_D, _NE + 2), lambda i: (0, 0)),
        ],
        out_specs=[
            pl.BlockSpec((tb, 6), lambda i: (i, 0)),
            pl.BlockSpec((tb, 6), lambda i: (i, 0)),
            pl.BlockSpec((1, 1), lambda i: (0, 0)),
        ],
        out_shape=[
            jax.ShapeDtypeStruct((n, 6), jnp.float32),
            jax.ShapeDtypeStruct((n, 6), jnp.int32),
            jax.ShapeDtypeStruct((1, 1), jnp.float32),
        ],
        scratch_shapes=[pltpu.VMEM((8, 128), jnp.float32)],
        compiler_params=pltpu.CompilerParams(
            dimension_semantics=("arbitrary",)),
    )(xf, w)

    return (rw.reshape(_B, _T, 6), idx.reshape(_B, _T, 6), aux[0, 0])


def kernel(x, W_expert, W_group):
    return _run(x, W_expert, W_group)
